# Initial kernel scaffold; baseline (speedup 1.0000x reference)
import jax
import jax.numpy as jnp
from jax.experimental import pallas as pl


def kernel(h, edge_index, W0, b0, W1, b1, W2, b2):
    raise NotImplementedError("write your pallas kernel here")


# speedup vs baseline: 8.0002x; 8.0002x over previous
import functools

import jax
import jax.numpy as jnp
from jax import lax
from jax.experimental import pallas as pl
from jax.experimental.pallas import tpu as pltpu
from jax.experimental.pallas import tpu_sc as plsc

N = 10000
E = 320000
D = 128
C = 7

NP = 10112
EP = 323584
NW = 32
EPW = EP // NW
CHUNK = 128
CPW = EPW // CHUNK
RPT = NP // 16
FLAT = NP * 8 // 128

_f32 = jnp.float32

_mesh = plsc.VectorSubcoreMesh(
    core_axis_name="c", subcore_axis_name="s", num_cores=2, num_subcores=16
)

_sc_params = pltpu.CompilerParams(use_tc_tiling_on_sc=False)


@functools.partial(
    pl.kernel,
    out_type=(
        jax.ShapeDtypeStruct((2, NP, 8), _f32),
        jax.ShapeDtypeStruct((2, NP, 8), _f32),
    ),
    mesh=_mesh,
    scratch_types=[
        pltpu.VMEM((CHUNK,), jnp.int32),
        pltpu.VMEM((CHUNK, 8), _f32),
        pltpu.VMEM_SHARED((NP, 8), _f32),
        pltpu.VMEM_SHARED((NP, 8), _f32),
    ],
    compiler_params=_sc_params,
)
def _deg_kernel(srcp, dstp, ones8, zeros8, outs, outd, idx_v, ones_v, accs, accd):
    cid = lax.axis_index("c")
    sid = lax.axis_index("s")
    wid = cid * 16 + sid
    pltpu.sync_copy(ones8, ones_v)
    pltpu.sync_copy(zeros8.at[pl.ds(sid * RPT, RPT)], accs.at[pl.ds(sid * RPT, RPT)])
    pltpu.sync_copy(zeros8.at[pl.ds(sid * RPT, RPT)], accd.at[pl.ds(sid * RPT, RPT)])
    plsc.subcore_barrier()

    def body(c, carry):
        base = wid * EPW + c * CHUNK
        pltpu.sync_copy(srcp.at[pl.ds(base, CHUNK)], idx_v)
        pltpu.sync_copy(ones_v, accs.at[idx_v], add=True)
        pltpu.sync_copy(dstp.at[pl.ds(base, CHUNK)], idx_v)
        pltpu.sync_copy(ones_v, accd.at[idx_v], add=True)
        return carry

    lax.fori_loop(0, CPW, body, 0)
    plsc.subcore_barrier()
    pltpu.sync_copy(accs.at[pl.ds(sid * RPT, RPT)], outs.at[cid, pl.ds(sid * RPT, RPT)])
    pltpu.sync_copy(accd.at[pl.ds(sid * RPT, RPT)], outd.at[cid, pl.ds(sid * RPT, RPT)])


@functools.partial(
    pl.kernel,
    out_type=jax.ShapeDtypeStruct((2, NP, 8), _f32),
    mesh=_mesh,
    scratch_types=[
        pltpu.VMEM((CHUNK,), jnp.int32),
        pltpu.VMEM((CHUNK,), jnp.int32),
        pltpu.VMEM((CHUNK, 8), _f32),
        pltpu.VMEM_SHARED((NP, 8), _f32),
        pltpu.SemaphoreType.DMA,
    ],
    compiler_params=_sc_params,
)
def _agg_kernel(x, srcp, dstp, zeros8, out, idxs_v, idxd_v, rows_v, acc, sem):
    cid = lax.axis_index("c")
    sid = lax.axis_index("s")
    wid = cid * 16 + sid
    pltpu.sync_copy(zeros8.at[pl.ds(sid * RPT, RPT)], acc.at[pl.ds(sid * RPT, RPT)])
    plsc.subcore_barrier()

    def body(c, carry):
        base = wid * EPW + c * CHUNK
        pltpu.sync_copy(srcp.at[pl.ds(base, CHUNK)], idxs_v)
        pltpu.sync_copy(dstp.at[pl.ds(base, CHUNK)], idxd_v)
        pltpu.async_copy(x.at[idxs_v], rows_v, sem).wait()
        pltpu.sync_copy(rows_v, acc.at[idxd_v], add=True)
        return carry

    lax.fori_loop(0, CPW, body, 0)
    plsc.subcore_barrier()
    pltpu.sync_copy(acc.at[pl.ds(sid * RPT, RPT)], out.at[cid, pl.ds(sid * RPT, RPT)])


def _proj_body(h_ref, w_ref, o_ref):
    o_ref[...] = jnp.dot(h_ref[...], w_ref[...], preferred_element_type=_f32)


def _prep_body(p_ref, ds0, ds1, dd0, dd1, x0, nsr, ndr):
    ns = lax.rsqrt(jnp.maximum(ds0[...] + ds1[...], 1.0))
    nd = lax.rsqrt(jnp.maximum(dd0[...] + dd1[...], 1.0))
    x0[...] = p_ref[...] * ns
    nsr[...] = ns
    ndr[...] = nd


def _mid1_body(a0, a1, ns, nd, br, x1):
    h1 = (a0[...] + a1[...]) * nd[...] + br[0:1, :]
    x1[...] = h1 * ns[...]


def _mid2_body(a0, a1, m_ref, ns, nd, br, x2):
    t = jnp.dot(a0[...] + a1[...], m_ref[...], preferred_element_type=_f32)
    x2[...] = (t * nd[...] + br[0:1, :]) * ns[...]


def _fin_body(a0, a1, m_ref, nd, br, o_ref):
    t = jnp.dot(a0[...] + a1[...], m_ref[...], preferred_element_type=_f32)
    o_ref[...] = t * nd[...] + br[0:1, :]


def _flat_sds(n=1):
    s = jax.ShapeDtypeStruct((FLAT, 128), _f32)
    return s if n == 1 else (s,) * n


def kernel(h, edge_index, W0, b0, W1, b1, W2, b2):
    src = edge_index[0]
    dst = edge_index[1]
    padi = N + (jnp.arange(EP - E, dtype=jnp.int32) % (NP - N))
    srcp = jnp.concatenate([src, padi])
    dstp = jnp.concatenate([dst, padi])

    hp = jnp.pad(h, ((0, NP - N), (0, 0)))
    W0p = jnp.pad(W0, ((0, 0), (0, 1)))
    W1p = jnp.pad(W1, ((0, 1), (0, 1)))
    W2p = jnp.pad(W2, ((0, 1), (0, 1)))
    eye16 = jnp.eye(16, dtype=_f32)
    M1 = jnp.kron(eye16, W1p)
    M2 = jnp.kron(eye16, W2p)
    b0r = jnp.broadcast_to(jnp.tile(jnp.pad(b0, (0, 1)), 16), (8, 128))
    b1r = jnp.broadcast_to(jnp.tile(jnp.pad(b1, (0, 1)), 16), (8, 128))
    b2r = jnp.broadcast_to(jnp.tile(jnp.pad(b2, (0, 1)), 16), (8, 128))
    zeros8 = jnp.zeros((NP, 8), _f32)
    ones8 = jnp.ones((CHUNK, 8), _f32)

    degs, degd = _deg_kernel(srcp, dstp, ones8, zeros8)
    p = pl.pallas_call(
        _proj_body, out_shape=jax.ShapeDtypeStruct((NP, 8), _f32)
    )(hp, W0p)

    pf = p.reshape(FLAT, 128)
    dsf = degs.reshape(2, FLAT, 128)
    ddf = degd.reshape(2, FLAT, 128)
    x0f, nsf, ndf = pl.pallas_call(_prep_body, out_shape=_flat_sds(3))(
        pf, dsf[0], dsf[1], ddf[0], ddf[1]
    )

    a1 = _agg_kernel(x0f.reshape(NP, 8), srcp, dstp, zeros8).reshape(2, FLAT, 128)
    x1f = pl.pallas_call(_mid1_body, out_shape=_flat_sds())(
        a1[0], a1[1], nsf, ndf, b0r
    )

    a2 = _agg_kernel(x1f.reshape(NP, 8), srcp, dstp, zeros8).reshape(2, FLAT, 128)
    x2f = pl.pallas_call(_mid2_body, out_shape=_flat_sds())(
        a2[0], a2[1], M1, nsf, ndf, b1r
    )

    a3 = _agg_kernel(x2f.reshape(NP, 8), srcp, dstp, zeros8).reshape(2, FLAT, 128)
    of = pl.pallas_call(_fin_body, out_shape=_flat_sds())(
        a3[0], a3[1], M2, ndf, b2r
    )
    return of.reshape(NP, 8)[:N, :C]


# speedup vs baseline: 23.8707x; 2.9838x over previous
import functools

import jax
import jax.numpy as jnp
from jax import lax
from jax.experimental import pallas as pl
from jax.experimental.pallas import tpu as pltpu
from jax.experimental.pallas import tpu_sc as plsc

N = 10000
E = 320000
D = 128
C = 7

NP = 10112
NW = 32
EPW = 10240
EP = NW * EPW
CHUNK = 512
CPW = EPW // CHUNK
RPT = NP // 16
FLAT = NP * 8 // 128

_f32 = jnp.float32

_mesh = plsc.VectorSubcoreMesh(
    core_axis_name="c", subcore_axis_name="s", num_cores=2, num_subcores=16
)
_sc_params = pltpu.CompilerParams(use_tc_tiling_on_sc=False)


@functools.partial(
    pl.kernel,
    out_type=(
        jax.ShapeDtypeStruct((2, NP, 8), _f32),
        jax.ShapeDtypeStruct((2, NP, 8), _f32),
    ),
    mesh=_mesh,
    scratch_types=[
        pltpu.VMEM((CPW, CHUNK), jnp.int32),
        pltpu.VMEM((CPW, CHUNK), jnp.int32),
        pltpu.VMEM((CHUNK, 8), _f32),
        pltpu.VMEM_SHARED((NP, 8), _f32),
        pltpu.VMEM_SHARED((NP, 8), _f32),
    ],
    compiler_params=_sc_params,
)
def _deg_kernel(src3, dst3, ones8, zeros8, outs, outd, ixs, ixd, ones_v, accs, accd):
    cid = lax.axis_index("c")
    sid = lax.axis_index("s")
    wid = cid * 16 + sid
    sl = pl.ds(sid * RPT, RPT)
    pltpu.sync_copy(ones8, ones_v)
    pltpu.sync_copy(zeros8.at[sl], accs.at[sl])
    pltpu.sync_copy(zeros8.at[sl], accd.at[sl])
    pltpu.sync_copy(src3.at[wid], ixs)
    pltpu.sync_copy(dst3.at[wid], ixd)
    plsc.subcore_barrier()

    def body(c, carry):
        pltpu.sync_copy(ones_v, accs.at[ixs.at[c]], add=True)
        pltpu.sync_copy(ones_v, accd.at[ixd.at[c]], add=True)
        return carry

    lax.fori_loop(0, CPW, body, 0)
    plsc.subcore_barrier()
    pltpu.sync_copy(accs.at[sl], outs.at[cid, sl])
    pltpu.sync_copy(accd.at[sl], outd.at[cid, sl])


@functools.partial(
    pl.kernel,
    out_type=jax.ShapeDtypeStruct((2, NP, 8), _f32),
    mesh=_mesh,
    scratch_types=[
        pltpu.VMEM((CPW, CHUNK), jnp.int32),
        pltpu.VMEM((CPW, CHUNK), jnp.int32),
        pltpu.VMEM((CHUNK, 8), _f32),
        pltpu.VMEM((CHUNK, 8), _f32),
        pltpu.VMEM_SHARED((NP, 8), _f32),
        pltpu.SemaphoreType.DMA,
        pltpu.SemaphoreType.DMA,
    ],
    compiler_params=_sc_params,
)
def _agg_kernel(x, src3, dst3, zeros8, out, ixs, ixd, r0, r1, acc, s0, s1):
    cid = lax.axis_index("c")
    sid = lax.axis_index("s")
    wid = cid * 16 + sid
    sl = pl.ds(sid * RPT, RPT)
    pltpu.sync_copy(zeros8.at[sl], acc.at[sl])
    pltpu.sync_copy(src3.at[wid], ixs)
    pltpu.sync_copy(dst3.at[wid], ixd)
    plsc.subcore_barrier()

    pltpu.async_copy(x.at[ixs.at[0]], r0, s0)
    pltpu.async_copy(x.at[ixs.at[1]], r1, s1)

    def body(t, carry):
        c = 2 * t
        pltpu.make_async_copy(x.at[ixs.at[c]], r0, s0).wait()
        pltpu.sync_copy(r0, acc.at[ixd.at[c]], add=True)
        pltpu.async_copy(x.at[ixs.at[c + 2]], r0, s0)
        pltpu.make_async_copy(x.at[ixs.at[c + 1]], r1, s1).wait()
        pltpu.sync_copy(r1, acc.at[ixd.at[c + 1]], add=True)
        pltpu.async_copy(x.at[ixs.at[c + 3]], r1, s1)
        return carry

    lax.fori_loop(0, CPW // 2 - 1, body, 0)
    c = CPW - 2
    pltpu.make_async_copy(x.at[ixs.at[c]], r0, s0).wait()
    pltpu.sync_copy(r0, acc.at[ixd.at[c]], add=True)
    pltpu.make_async_copy(x.at[ixs.at[c + 1]], r1, s1).wait()
    pltpu.sync_copy(r1, acc.at[ixd.at[c + 1]], add=True)

    plsc.subcore_barrier()
    pltpu.sync_copy(acc.at[sl], out.at[cid, sl])


def _proj_body(h_ref, w_ref, o_ref):
    o_ref[...] = jnp.dot(h_ref[...], w_ref[...], preferred_element_type=_f32)


def _prep_body(p_ref, ds0, ds1, dd0, dd1, x0, nsr, ndr):
    ns = lax.rsqrt(jnp.maximum(ds0[...] + ds1[...], 1.0))
    nd = lax.rsqrt(jnp.maximum(dd0[...] + dd1[...], 1.0))
    x0[...] = p_ref[...] * ns
    nsr[...] = ns
    ndr[...] = nd


def _mid1_body(a0, a1, ns, nd, br, x1):
    h1 = (a0[...] + a1[...]) * nd[...] + br[0:1, :]
    x1[...] = h1 * ns[...]


def _mid2_body(a0, a1, m_ref, ns, nd, br, x2):
    t = jnp.dot(a0[...] + a1[...], m_ref[...], preferred_element_type=_f32)
    x2[...] = (t * nd[...] + br[0:1, :]) * ns[...]


def _fin_body(a0, a1, m_ref, nd, br, o_ref):
    t = jnp.dot(a0[...] + a1[...], m_ref[...], preferred_element_type=_f32)
    o_ref[...] = t * nd[...] + br[0:1, :]


def _flat_sds(n=1):
    s = jax.ShapeDtypeStruct((FLAT, 128), _f32)
    return s if n == 1 else (s,) * n


def kernel(h, edge_index, W0, b0, W1, b1, W2, b2):
    src = edge_index[0]
    dst = edge_index[1]
    padi = N + (jnp.arange(EP - E, dtype=jnp.int32) % (NP - N))
    src3 = jnp.concatenate([src, padi]).reshape(NW, CPW, CHUNK)
    dst3 = jnp.concatenate([dst, padi]).reshape(NW, CPW, CHUNK)

    hp = jnp.pad(h, ((0, NP - N), (0, 0)))
    W0p = jnp.pad(W0, ((0, 0), (0, 1)))
    W1p = jnp.pad(W1, ((0, 1), (0, 1)))
    W2p = jnp.pad(W2, ((0, 1), (0, 1)))
    eye16 = jnp.eye(16, dtype=_f32)
    M1 = jnp.kron(eye16, W1p)
    M2 = jnp.kron(eye16, W2p)
    b0r = jnp.broadcast_to(jnp.tile(jnp.pad(b0, (0, 1)), 16), (8, 128))
    b1r = jnp.broadcast_to(jnp.tile(jnp.pad(b1, (0, 1)), 16), (8, 128))
    b2r = jnp.broadcast_to(jnp.tile(jnp.pad(b2, (0, 1)), 16), (8, 128))
    zeros8 = jnp.zeros((NP, 8), _f32)
    ones8 = jnp.ones((CHUNK, 8), _f32)

    degs, degd = _deg_kernel(src3, dst3, ones8, zeros8)
    p = pl.pallas_call(
        _proj_body, out_shape=jax.ShapeDtypeStruct((NP, 8), _f32)
    )(hp, W0p)

    pf = p.reshape(FLAT, 128)
    dsf = degs.reshape(2, FLAT, 128)
    ddf = degd.reshape(2, FLAT, 128)
    x0f, nsf, ndf = pl.pallas_call(_prep_body, out_shape=_flat_sds(3))(
        pf, dsf[0], dsf[1], ddf[0], ddf[1]
    )

    a1 = _agg_kernel(x0f.reshape(NP, 8), src3, dst3, zeros8).reshape(2, FLAT, 128)
    x1f = pl.pallas_call(_mid1_body, out_shape=_flat_sds())(
        a1[0], a1[1], nsf, ndf, b0r
    )

    a2 = _agg_kernel(x1f.reshape(NP, 8), src3, dst3, zeros8).reshape(2, FLAT, 128)
    x2f = pl.pallas_call(_mid2_body, out_shape=_flat_sds())(
        a2[0], a2[1], M1, nsf, ndf, b1r
    )

    a3 = _agg_kernel(x2f.reshape(NP, 8), src3, dst3, zeros8).reshape(2, FLAT, 128)
    of = pl.pallas_call(_fin_body, out_shape=_flat_sds())(
        a3[0], a3[1], M2, ndf, b2r
    )
    return of.reshape(NP, 8)[:N, :C]
